# Initial kernel scaffold; baseline (speedup 1.0000x reference)
import jax
import jax.numpy as jnp
from jax.experimental import pallas as pl


def kernel(word_idx, context_idx, table, W, b):
    raise NotImplementedError("write your pallas kernel here")


# speedup vs baseline: 93.6749x; 93.6749x over previous
import functools

import jax
import jax.numpy as jnp
from jax import lax
from jax.experimental import pallas as pl
from jax.experimental.pallas import tpu as pltpu
from jax.experimental.pallas import tpu_sc as plsc

B = 16384
L = 200
VOCAB = 22
D = 128
VP = 24

NC = 2
NS = 16
NW = NC * NS
RPW = B // NW
CH = 128
NCHUNK = RPW // CH
SUB = CH // 16


@functools.cache
def _sc_counts_fn():
    return functools.partial(
        pl.kernel,
        out_type=jax.ShapeDtypeStruct((B * VP,), jnp.float32),
        mesh=plsc.VectorSubcoreMesh(
            core_axis_name="c", subcore_axis_name="s", num_cores=NC, num_subcores=NS
        ),
        scratch_types=[
            pltpu.VMEM((CH * L,), jnp.int32),
            pltpu.VMEM((CH * VP,), jnp.float32),
        ],
        compiler_params=pltpu.CompilerParams(needs_layout_passes=False),
    )(_sc_counts)


def _sc_counts(ctx_hbm, out_hbm, ctx_buf, cnt_buf):
    wid = lax.axis_index("s") * NC + lax.axis_index("c")
    lane = lax.iota(jnp.int32, 16)
    ones = jnp.ones((16,), jnp.float32)
    zeros16 = jnp.zeros((16,), jnp.float32)
    gbase = [(lane + s * 16) * L for s in range(SUB)]
    abase = [(lane + s * 16) * VP for s in range(SUB)]

    for c in range(NCHUNK):
        row0 = wid * RPW + c * CH
        pltpu.sync_copy(ctx_hbm.at[pl.ds(row0 * L, CH * L)], ctx_buf)

        def zbody(i, carry):
            cnt_buf[pl.ds(i * 16, 16)] = zeros16
            return carry

        lax.fori_loop(0, CH * VP // 16, zbody, 0)

        def body(l, carry):
            for s in range(SUB):
                v = plsc.load_gather(ctx_buf, [gbase[s] + l])
                plsc.addupdate_scatter(cnt_buf, [abase[s] + v], ones)
            return carry

        lax.fori_loop(0, L, body, 0)
        pltpu.sync_copy(cnt_buf, out_hbm.at[pl.ds(row0 * VP, CH * VP)])


NB = 1024


def _tc_body(cnt_ref, word_ref, tab_ref, w_ref, b_ref, out_ref):
    t1 = jnp.dot(tab_ref[...], w_ref[0:D, :], preferred_element_type=jnp.float32)
    t2 = jnp.dot(tab_ref[...], w_ref[D:, :], preferred_element_type=jnp.float32)
    woh = (word_ref[...] == lax.broadcasted_iota(jnp.int32, (NB, VP), 1)).astype(
        jnp.float32
    )
    ctxm = cnt_ref[...] * (1.0 / L)
    out_ref[...] = (
        jnp.dot(woh, t1, preferred_element_type=jnp.float32)
        + jnp.dot(ctxm, t2, preferred_element_type=jnp.float32)
        + b_ref[...]
    )


def _tc_call(counts, word2, tablep, W, b2):
    return pl.pallas_call(
        _tc_body,
        grid=(B // NB,),
        in_specs=[
            pl.BlockSpec((NB, VP), lambda i: (i, 0)),
            pl.BlockSpec((NB, 1), lambda i: (i, 0)),
            pl.BlockSpec((VP, D), lambda i: (0, 0)),
            pl.BlockSpec((2 * D, D), lambda i: (0, 0)),
            pl.BlockSpec((1, D), lambda i: (0, 0)),
        ],
        out_specs=pl.BlockSpec((NB, D), lambda i: (i, 0)),
        out_shape=jax.ShapeDtypeStruct((B, D), jnp.float32),
    )(counts, word2, tablep, W, b2)


def kernel(word_idx, context_idx, table, W, b):
    counts = _sc_counts_fn()(context_idx.reshape(-1))
    tablep = jnp.concatenate(
        [table, jnp.zeros((VP - VOCAB, D), jnp.float32)], axis=0
    )
    return _tc_call(
        counts.reshape(B, VP),
        word_idx.reshape(B, 1),
        tablep,
        W,
        b.reshape(1, D),
    )


# speedup vs baseline: 131.6466x; 1.4054x over previous
import functools

import jax
import jax.numpy as jnp
from jax import lax
from jax.experimental import pallas as pl
from jax.experimental.pallas import tpu as pltpu
from jax.experimental.pallas import tpu_sc as plsc

B = 16384
L = 200
VOCAB = 22
D = 128
VP = 24

NC = 2
NS = 16
NW = NC * NS
RPW = B // NW
CH = 128
NCHUNK = RPW // CH
SUB = CH // 16


def _sc_counts(ctx_hbm, out_hbm, ctx_buf, cnt_buf):
    wid = lax.axis_index("s") * NC + lax.axis_index("c")
    lane = lax.iota(jnp.int32, 16)
    ones = jnp.ones((16,), jnp.float32)
    zeros16 = jnp.zeros((16,), jnp.float32)
    rows = [lane + s * 16 for s in range(SUB)]

    for c in range(NCHUNK):
        row0 = wid * RPW + c * CH
        pltpu.sync_copy(ctx_hbm.at[pl.ds(row0, CH), :], ctx_buf)

        def zbody(r, carry):
            cnt_buf[r, pl.ds(0, 16)] = zeros16
            cnt_buf[r, pl.ds(VP - 16, 16)] = zeros16
            return carry

        lax.fori_loop(0, CH, zbody, 0)

        @plsc.parallel_loop(0, L, unroll=4)
        def _(l):
            col = jnp.full((16,), 0, jnp.int32) + l
            for s in range(SUB):
                v = plsc.load_gather(ctx_buf, [rows[s], col])
                plsc.addupdate_scatter(cnt_buf, [rows[s], v], ones)

        pltpu.sync_copy(cnt_buf, out_hbm.at[pl.ds(row0, CH), :])


@functools.cache
def _sc_counts_fn():
    return functools.partial(
        pl.kernel,
        out_type=jax.ShapeDtypeStruct((B, VP), jnp.float32),
        mesh=plsc.VectorSubcoreMesh(
            core_axis_name="c", subcore_axis_name="s", num_cores=NC, num_subcores=NS
        ),
        scratch_types=[
            pltpu.VMEM((CH, L), jnp.int32),
            pltpu.VMEM((CH, VP), jnp.float32),
        ],
        compiler_params=pltpu.CompilerParams(needs_layout_passes=False),
    )(_sc_counts)


NB = 2048


def _tc_body(cnt_ref, word_ref, tab_ref, w_ref, b_ref, out_ref, t1_ref, t2_ref):
    @pl.when(pl.program_id(0) == 0)
    def _():
        t1_ref[...] = jnp.dot(
            tab_ref[...], w_ref[0:D, :], preferred_element_type=jnp.float32
        )
        t2_ref[...] = jnp.dot(
            tab_ref[...], w_ref[D:, :], preferred_element_type=jnp.float32
        )

    woh = (word_ref[...] == lax.broadcasted_iota(jnp.int32, (NB, VP), 1)).astype(
        jnp.float32
    )
    ctxm = cnt_ref[...] * (1.0 / L)
    out_ref[...] = (
        jnp.dot(woh, t1_ref[...], preferred_element_type=jnp.float32)
        + jnp.dot(ctxm, t2_ref[...], preferred_element_type=jnp.float32)
        + b_ref[...]
    )


def _tc_call(counts, word2, tablep, W, b2):
    return pl.pallas_call(
        _tc_body,
        grid=(B // NB,),
        in_specs=[
            pl.BlockSpec((NB, VP), lambda i: (i, 0)),
            pl.BlockSpec((NB, 1), lambda i: (i, 0)),
            pl.BlockSpec((VP, D), lambda i: (0, 0)),
            pl.BlockSpec((2 * D, D), lambda i: (0, 0)),
            pl.BlockSpec((1, D), lambda i: (0, 0)),
        ],
        out_specs=pl.BlockSpec((NB, D), lambda i: (i, 0)),
        out_shape=jax.ShapeDtypeStruct((B, D), jnp.float32),
        scratch_shapes=[
            pltpu.VMEM((VP, D), jnp.float32),
            pltpu.VMEM((VP, D), jnp.float32),
        ],
    )(counts, word2, tablep, W, b2)


def kernel(word_idx, context_idx, table, W, b):
    counts = _sc_counts_fn()(context_idx)
    tablep = jnp.concatenate(
        [table, jnp.zeros((VP - VOCAB, D), jnp.float32)], axis=0
    )
    return _tc_call(
        counts,
        word_idx.reshape(B, 1),
        tablep,
        W,
        b.reshape(1, D),
    )
